# Initial kernel scaffold; baseline (speedup 1.0000x reference)
import jax
import jax.numpy as jnp
from jax.experimental import pallas as pl


def kernel(data, user_factors, item_factors):
    raise NotImplementedError("write your pallas kernel here")


# speedup vs baseline: 4.8647x; 4.8647x over previous
import functools

import jax
import jax.numpy as jnp
from jax import lax
from jax.experimental import pallas as pl
from jax.experimental.pallas import tpu as pltpu
from jax.experimental.pallas import tpu_sc as plsc

NC, NS, L = 2, 16, 16
NW = NC * NS
B = 16384
BPW = B // NW
NV = BPW // L
U_ROWS, I_ROWS, D = 1500, 2000, 3

_mesh = plsc.VectorSubcoreMesh(core_axis_name="c", subcore_axis_name="s")


@functools.partial(
    pl.kernel,
    out_type=jax.ShapeDtypeStruct((B,), jnp.float32),
    mesh=_mesh,
    scratch_types=[
        pltpu.VMEM((BPW,), jnp.int32),
        pltpu.VMEM((BPW,), jnp.int32),
        pltpu.VMEM((U_ROWS * D,), jnp.float32),
        pltpu.VMEM((I_ROWS * D,), jnp.float32),
        pltpu.VMEM((BPW,), jnp.float32),
        pltpu.SemaphoreType.DMA,
    ],
    compiler_params=pltpu.CompilerParams(needs_layout_passes=False),
)
def _mf_kernel(uidx_hbm, iidx_hbm, utab_hbm, itab_hbm, out_hbm,
               uidx_v, iidx_v, utab_v, itab_v, out_v, sem):
    wid = lax.axis_index("s") * NC + lax.axis_index("c")
    base = wid * BPW

    c1 = pltpu.make_async_copy(uidx_hbm.at[pl.ds(base, BPW)], uidx_v, sem)
    c2 = pltpu.make_async_copy(iidx_hbm.at[pl.ds(base, BPW)], iidx_v, sem)
    c3 = pltpu.make_async_copy(utab_hbm, utab_v, sem)
    c4 = pltpu.make_async_copy(itab_hbm, itab_v, sem)
    c1.start()
    c2.start()
    c3.start()
    c4.start()
    c1.wait()
    c2.wait()
    c3.wait()
    c4.wait()

    three = jnp.full((L,), D, jnp.int32)
    for i in range(NV):
        ub = uidx_v[pl.ds(i * L, L)] * three
        vb = iidx_v[pl.ds(i * L, L)] * three
        acc = None
        for d in range(D):
            uu = plsc.load_gather(utab_v, [ub + d])
            vv = plsc.load_gather(itab_v, [vb + d])
            prod = uu * vv
            acc = prod if acc is None else acc + prod
        out_v[pl.ds(i * L, L)] = acc

    pltpu.sync_copy(out_v, out_hbm.at[pl.ds(base, BPW)])


def kernel(data, user_factors, item_factors):
    data = data.astype(jnp.int32)
    return _mf_kernel(data[0], data[1],
                      user_factors.reshape(-1), item_factors.reshape(-1))


# speedup vs baseline: 5.0579x; 1.0397x over previous
import functools

import jax
import jax.numpy as jnp
from jax import lax
from jax.experimental import pallas as pl
from jax.experimental.pallas import tpu as pltpu
from jax.experimental.pallas import tpu_sc as plsc

NC, NS, L = 2, 16, 16
NW = NC * NS
B = 16384
BPW = B // NW
NV = BPW // L
U_ROWS, I_ROWS, D = 1500, 2000, 3

_mesh = plsc.VectorSubcoreMesh(core_axis_name="c", subcore_axis_name="s")


@functools.partial(
    pl.kernel,
    out_type=jax.ShapeDtypeStruct((B,), jnp.float32),
    mesh=_mesh,
    scratch_types=[
        pltpu.VMEM((BPW,), jnp.int32),
        pltpu.VMEM((BPW,), jnp.int32),
        pltpu.VMEM((U_ROWS * D,), jnp.float32),
        pltpu.VMEM((I_ROWS * D,), jnp.float32),
        pltpu.VMEM((BPW,), jnp.float32),
        pltpu.SemaphoreType.DMA,
    ],
    compiler_params=pltpu.CompilerParams(
        needs_layout_passes=False, skip_device_barrier=True),
)
def _mf_kernel(data_hbm, utab_hbm, itab_hbm, out_hbm,
               uidx_v, iidx_v, utab_v, itab_v, out_v, sem):
    wid = lax.axis_index("s") * NC + lax.axis_index("c")
    base = wid * BPW

    c1 = pltpu.make_async_copy(data_hbm.at[0, pl.ds(base, BPW)], uidx_v, sem)
    c2 = pltpu.make_async_copy(data_hbm.at[1, pl.ds(base, BPW)], iidx_v, sem)
    c3 = pltpu.make_async_copy(utab_hbm, utab_v, sem)
    c4 = pltpu.make_async_copy(itab_hbm, itab_v, sem)
    c1.start()
    c2.start()
    c3.start()
    c4.start()
    c1.wait()
    c2.wait()
    c3.wait()
    c4.wait()

    three = jnp.full((L,), D, jnp.int32)
    for i in range(NV):
        ub = uidx_v[pl.ds(i * L, L)] * three
        vb = iidx_v[pl.ds(i * L, L)] * three
        acc = None
        for d in range(D):
            uu = plsc.load_gather(utab_v, [ub + d])
            vv = plsc.load_gather(itab_v, [vb + d])
            prod = uu * vv
            acc = prod if acc is None else acc + prod
        out_v[pl.ds(i * L, L)] = acc

    pltpu.sync_copy(out_v, out_hbm.at[pl.ds(base, BPW)])


def kernel(data, user_factors, item_factors):
    data = data.astype(jnp.int32)
    return _mf_kernel(data,
                      user_factors.reshape(-1), item_factors.reshape(-1))
